# baseline (device time: 75690 ns/iter reference)
import jax
import jax.numpy as jnp
from jax import lax
from jax.experimental import pallas as pl
from jax.experimental.pallas import tpu as pltpu

B, H, D, BS = 16, 16, 64, 16
NB = 128
PAGES = 128
TOK = PAGES * BS
HB = H * B
HD = H * D
NEG = -1e30
SCALE = D ** -0.5


def kernel(Q, K, V, bt, lens):
    QT = jnp.transpose(Q[:, 0, :, :].reshape(B, HD))
    Kt = K.reshape(TOK, HD)
    Vt = V.reshape(TOK, HD)
    lens2 = lens[:, None]

    def body(q_ref, k_ref, v_ref, bt_ref, lens_ref, out_ref,
             o_com, ml_com, send_sems, recv_sems):
        my_x = lax.axis_index("x")
        my_y = lax.axis_index("y")
        nbr = (1 - my_x, my_y)

        barrier = pltpu.get_barrier_semaphore()
        pl.semaphore_signal(barrier, inc=1, device_id=nbr,
                            device_id_type=pl.DeviceIdType.MESH)
        pl.semaphore_wait(barrier, 1)

        bt_ = bt_ref[...]
        lens_ = lens_ref[...]

        j = lax.broadcasted_iota(jnp.int32, (B, NB), 1)
        btl = bt_ - my_x * PAGES
        ok = (j < lens_) & (btl >= 0) & (btl < PAGES)
        pg = lax.broadcasted_iota(jnp.int32, (PAGES, B, NB), 0)
        hit = (btl[None] == pg) & ok[None]
        counts_t = jnp.sum(jnp.where(hit, 1.0, 0.0), axis=2)

        ki = lax.broadcasted_iota(jnp.int32, (TOK, PAGES), 0)
        pi = lax.broadcasted_iota(jnp.int32, (TOK, PAGES), 1)
        e_t = jnp.where((ki // BS) == pi, 1.0, 0.0)
        w_t = lax.dot_general(e_t, counts_t, (((1,), (0,)), ((), ())),
                              preferred_element_type=jnp.float32)
        wmask = w_t > 0.0

        for h in range(H):
            kh = k_ref[:, pl.ds(h * D, D)]
            qh = q_ref[pl.ds(h * D, D)]
            s = lax.dot_general(kh, qh, (((1,), (0,)), ((), ())),
                                preferred_element_type=jnp.float32)
            sm = jnp.where(wmask, s * SCALE, NEG)
            m_h = jnp.max(sm, axis=0, keepdims=True)
            p = w_t * jnp.exp(sm - m_h)
            l_h = jnp.sum(p, axis=0, keepdims=True)
            o_h = lax.dot_general(p, v_ref[:, pl.ds(h * D, D)],
                                  (((0,), (0,)), ((), ())),
                                  preferred_element_type=jnp.float32)
            o_com[0, pl.ds(h * B, B)] = o_h
            ml_com[0, 0, :, pl.ds(h * B, B)] = m_h
            ml_com[0, 1, :, pl.ds(h * B, B)] = l_h

        rdma_o = pltpu.make_async_remote_copy(
            src_ref=o_com.at[0], dst_ref=o_com.at[1],
            send_sem=send_sems.at[0], recv_sem=recv_sems.at[0],
            device_id=nbr, device_id_type=pl.DeviceIdType.MESH)
        rdma_ml = pltpu.make_async_remote_copy(
            src_ref=ml_com.at[0], dst_ref=ml_com.at[1],
            send_sem=send_sems.at[1], recv_sem=recv_sems.at[1],
            device_id=nbr, device_id_type=pl.DeviceIdType.MESH)
        rdma_o.start()
        rdma_ml.start()
        rdma_o.wait()
        rdma_ml.wait()

        m0 = ml_com[0, 0]
        l0 = ml_com[0, 1]
        m1 = ml_com[1, 0]
        l1 = ml_com[1, 1]
        mg = jnp.maximum(m0, m1)
        s0 = jnp.exp(m0 - mg)
        s1 = jnp.exp(m1 - mg)
        lg = l0 * s0 + l1 * s1
        cat = jnp.concatenate([s0, s1, lg], axis=0)
        r0 = lax.broadcasted_iota(jnp.int32, (HB, HB), 0)
        r1 = lax.broadcasted_iota(jnp.int32, (HB, HB), 1)
        ident = jnp.where(r0 == r1, 1.0, 0.0)
        catT = lax.dot_general(ident, cat, (((1,), (1,)), ((), ())),
                               preferred_element_type=jnp.float32)
        s0c = catT[:, 0:1]
        s1c = catT[:, 1:2]
        lgc = catT[:, 2:3]
        out_ref[...] = (o_com[0] * s0c + o_com[1] * s1c) / lgc

    out = pl.pallas_call(
        body,
        out_shape=jax.ShapeDtypeStruct((HB, D), jnp.float32),
        in_specs=[pl.BlockSpec(memory_space=pltpu.VMEM)] * 5,
        out_specs=pl.BlockSpec(memory_space=pltpu.VMEM),
        scratch_shapes=[
            pltpu.VMEM((2, HB, D), jnp.float32),
            pltpu.VMEM((2, 2, 1, HB), jnp.float32),
            pltpu.SemaphoreType.DMA((2,)),
            pltpu.SemaphoreType.DMA((2,)),
        ],
        compiler_params=pltpu.CompilerParams(
            collective_id=0, vmem_limit_bytes=56 * 1024 * 1024),
    )(QT, Kt, Vt, bt, lens2)

    return jnp.transpose(out.reshape(H, B, D), (1, 0, 2))[:, None, :, :]


# device time: 22788 ns/iter; 3.3215x vs baseline; 3.3215x over previous
import jax
import jax.numpy as jnp
from jax import lax
from jax.experimental import pallas as pl
from jax.experimental.pallas import tpu as pltpu

B, H, D, BS = 16, 16, 64, 16
NB = 128
PAGES = 128
TOK = PAGES * BS
HB = H * B
NEG = -1e30
SCALE = D ** -0.5


def kernel(Q, K, V, bt, lens):
    Qh = jnp.transpose(Q[:, 0, :, :], (1, 0, 2))
    Kp = jnp.transpose(K, (1, 2, 3, 0))
    Vp = jnp.transpose(V, (1, 2, 3, 0))
    lens2 = lens[:, None]

    def body(q_ref, k_ref, v_ref, bt_ref, lens_ref, out_ref,
             o_com, ml_com, send_sems, recv_sems):
        my_x = lax.axis_index("x")
        my_y = lax.axis_index("y")
        nbr = (1 - my_x, my_y)

        barrier = pltpu.get_barrier_semaphore()
        pl.semaphore_signal(barrier, inc=1, device_id=nbr,
                            device_id_type=pl.DeviceIdType.MESH)
        pl.semaphore_wait(barrier, 1)

        bt_ = bt_ref[...]
        lens_ = lens_ref[...]

        j = lax.broadcasted_iota(jnp.int32, (B, NB), 1)
        btl = bt_ - my_x * PAGES
        ok = (j < lens_) & (btl >= 0) & (btl < PAGES)
        pg = lax.broadcasted_iota(jnp.int32, (B, PAGES, NB), 1)
        hit = (btl[:, None, :] == pg) & ok[:, None, :]
        counts = jnp.sum(jnp.where(hit, 1.0, 0.0), axis=2)

        w = jnp.concatenate([counts] * BS, axis=1)
        wmask = w > 0.0

        for h in range(H):
            khT = jnp.concatenate([k_ref[s, h] for s in range(BS)],
                                  axis=1)
            s = lax.dot_general(q_ref[h], khT, (((1,), (0,)), ((), ())),
                                preferred_element_type=jnp.float32)
            sm = jnp.where(wmask, s * SCALE, NEG)
            m_h = jnp.max(sm, axis=1, keepdims=True)
            p = w * jnp.exp(sm - m_h)
            l_h = jnp.sum(p, axis=1, keepdims=True)
            vhT = jnp.concatenate([v_ref[s, h] for s in range(BS)],
                                  axis=1)
            o_h = lax.dot_general(p, vhT, (((1,), (1,)), ((), ())),
                                  preferred_element_type=jnp.float32)
            o_com[0, pl.ds(h * B, B)] = o_h
            ml_com[0, 0, pl.ds(h * B, B)] = m_h
            ml_com[0, 1, pl.ds(h * B, B)] = l_h

        rdma_o = pltpu.make_async_remote_copy(
            src_ref=o_com.at[0], dst_ref=o_com.at[1],
            send_sem=send_sems.at[0], recv_sem=recv_sems.at[0],
            device_id=nbr, device_id_type=pl.DeviceIdType.MESH)
        rdma_ml = pltpu.make_async_remote_copy(
            src_ref=ml_com.at[0], dst_ref=ml_com.at[1],
            send_sem=send_sems.at[1], recv_sem=recv_sems.at[1],
            device_id=nbr, device_id_type=pl.DeviceIdType.MESH)
        rdma_o.start()
        rdma_ml.start()
        rdma_o.wait()
        rdma_ml.wait()

        m0 = ml_com[0, 0]
        l0 = ml_com[0, 1]
        m1 = ml_com[1, 0]
        l1 = ml_com[1, 1]
        mg = jnp.maximum(m0, m1)
        s0 = jnp.exp(m0 - mg)
        s1 = jnp.exp(m1 - mg)
        lg = l0 * s0 + l1 * s1
        out_ref[...] = (o_com[0] * s0 + o_com[1] * s1) / lg

    out = pl.pallas_call(
        body,
        out_shape=jax.ShapeDtypeStruct((HB, D), jnp.float32),
        in_specs=[pl.BlockSpec(memory_space=pltpu.VMEM)] * 5,
        out_specs=pl.BlockSpec(memory_space=pltpu.VMEM),
        scratch_shapes=[
            pltpu.VMEM((2, HB, D), jnp.float32),
            pltpu.VMEM((2, 2, HB, 1), jnp.float32),
            pltpu.SemaphoreType.DMA((2,)),
            pltpu.SemaphoreType.DMA((2,)),
        ],
        compiler_params=pltpu.CompilerParams(
            collective_id=0, vmem_limit_bytes=60 * 1024 * 1024),
    )(Qh, Kp, Vp, bt, lens2)

    return jnp.transpose(out.reshape(H, B, D), (1, 0, 2))[:, None, :, :]


# device time: 17369 ns/iter; 4.3578x vs baseline; 1.3120x over previous
import jax
import jax.numpy as jnp
from jax import lax
from jax.experimental import pallas as pl
from jax.experimental.pallas import tpu as pltpu

B, H, D, BS = 16, 16, 64, 16
NB = 128
PAGES = 128
TOK = PAGES * BS
HB = H * B
NEG = -1e30
SCALE = D ** -0.5


def kernel(Q, K, V, bt, lens):
    Q2 = Q[:, 0, :, :]
    Kp = jnp.transpose(K, (1, 2, 3, 0))
    Vp = jnp.transpose(V, (1, 2, 3, 0))
    lens1 = lens[None, :]

    def body(q_ref, k_hbm, v_hbm, bt_ref, lens_ref, out_ref,
             k_ref, v_ref, o_com, ml_com, og_vm, copy_sems, out_sem,
             send_sems, recv_sems):
        my_x = lax.axis_index("x")
        my_y = lax.axis_index("y")
        nbr = (1 - my_x, my_y)

        QH = 4
        kv_copies = []
        for q in range(H // QH):
            kc = pltpu.make_async_copy(
                k_hbm.at[:, pl.ds(q * QH, QH)],
                k_ref.at[:, pl.ds(q * QH, QH)], copy_sems.at[q])
            vc = pltpu.make_async_copy(
                v_hbm.at[:, pl.ds(q * QH, QH)],
                v_ref.at[:, pl.ds(q * QH, QH)], copy_sems.at[4 + q])
            kc.start()
            vc.start()
            kv_copies.append((kc, vc))

        barrier = pltpu.get_barrier_semaphore()
        pl.semaphore_signal(barrier, inc=1, device_id=nbr,
                            device_id_type=pl.DeviceIdType.MESH)

        bt_ = bt_ref[...]
        qh_all = jnp.transpose(q_ref[...], (1, 0, 2))

        r0 = lax.broadcasted_iota(jnp.int32, (B, B), 0)
        r1 = lax.broadcasted_iota(jnp.int32, (B, B), 1)
        ident16 = jnp.where(r0 == r1, 1.0, 0.0)
        lens_col = lax.dot_general(
            ident16, lens_ref[...].astype(jnp.float32),
            (((1,), (1,)), ((), ())),
            preferred_element_type=jnp.float32)

        lens_i = lens_col.astype(jnp.int32)
        j = lax.broadcasted_iota(jnp.int32, (B, NB), 1)
        btl = bt_ - my_x * PAGES
        ok = (j < lens_i) & (btl >= 0) & (btl < PAGES)
        pg = lax.broadcasted_iota(jnp.int32, (B, PAGES, NB), 1)
        hit = (btl[:, None, :] == pg) & ok[:, None, :]
        counts = jnp.sum(jnp.where(hit, 1.0, 0.0), axis=2)

        w = jnp.concatenate([counts] * BS, axis=1)
        wmask = w > 0.0

        def head_block(h):
            khT = jnp.concatenate([k_ref[s, h] for s in range(BS)],
                                  axis=1)
            s = lax.dot_general(qh_all[h], khT, (((1,), (0,)), ((), ())),
                                preferred_element_type=jnp.float32)
            sm = jnp.where(wmask, s * SCALE, NEG)
            m_h = jnp.max(sm, axis=1, keepdims=True)
            p = w * jnp.exp(sm - m_h)
            l_h = jnp.sum(p, axis=1, keepdims=True)
            vhT = jnp.concatenate([v_ref[s, h] for s in range(BS)],
                                  axis=1)
            o_h = lax.dot_general(p, vhT, (((1,), (1,)), ((), ())),
                                  preferred_element_type=jnp.float32)
            o_com[0, pl.ds(h * B, B)] = o_h
            ml_com[0, 0, pl.ds(h * B, B)] = m_h
            ml_com[0, 1, pl.ds(h * B, B)] = l_h

        HALF = H // 2 * B
        for h in range(H // 2):
            if h % 4 == 0:
                kv_copies[h // 4][0].wait()
                kv_copies[h // 4][1].wait()
            head_block(h)

        pl.semaphore_wait(barrier, 1)
        rdma_o1 = pltpu.make_async_remote_copy(
            src_ref=o_com.at[0, pl.ds(0, HALF)],
            dst_ref=o_com.at[1, pl.ds(0, HALF)],
            send_sem=send_sems.at[0], recv_sem=recv_sems.at[0],
            device_id=nbr, device_id_type=pl.DeviceIdType.MESH)
        rdma_o1.start()

        for h in range(H // 2, H):
            if h % 4 == 0:
                kv_copies[h // 4][0].wait()
                kv_copies[h // 4][1].wait()
            head_block(h)

        rdma_o2 = pltpu.make_async_remote_copy(
            src_ref=o_com.at[0, pl.ds(HALF, HALF)],
            dst_ref=o_com.at[1, pl.ds(HALF, HALF)],
            send_sem=send_sems.at[1], recv_sem=recv_sems.at[1],
            device_id=nbr, device_id_type=pl.DeviceIdType.MESH)
        rdma_ml = pltpu.make_async_remote_copy(
            src_ref=ml_com.at[0], dst_ref=ml_com.at[1],
            send_sem=send_sems.at[2], recv_sem=recv_sems.at[2],
            device_id=nbr, device_id_type=pl.DeviceIdType.MESH)
        rdma_o2.start()
        rdma_ml.start()
        rdma_o1.wait()
        rdma_o2.wait()
        rdma_ml.wait()

        m0 = ml_com[0, 0]
        l0 = ml_com[0, 1]
        m1 = ml_com[1, 0]
        l1 = ml_com[1, 1]
        mg = jnp.maximum(m0, m1)
        s0 = jnp.exp(m0 - mg)
        s1 = jnp.exp(m1 - mg)
        lg = l0 * s0 + l1 * s1
        og = (o_com[0] * s0 + o_com[1] * s1) / lg
        og_vm[...] = jnp.transpose(og.reshape(H, B, D), (1, 0, 2))
        out_copy = pltpu.make_async_copy(og_vm, out_ref, out_sem)
        out_copy.start()
        out_copy.wait()

    out = pl.pallas_call(
        body,
        out_shape=jax.ShapeDtypeStruct((B, H, D), jnp.float32),
        in_specs=[
            pl.BlockSpec(memory_space=pltpu.VMEM),
            pl.BlockSpec(memory_space=pl.ANY),
            pl.BlockSpec(memory_space=pl.ANY),
            pl.BlockSpec(memory_space=pltpu.VMEM),
            pl.BlockSpec(memory_space=pltpu.VMEM),
        ],
        out_specs=pl.BlockSpec(memory_space=pl.ANY),
        scratch_shapes=[
            pltpu.VMEM((BS, H, D, PAGES), jnp.float32),
            pltpu.VMEM((BS, H, D, PAGES), jnp.float32),
            pltpu.VMEM((2, HB, D), jnp.float32),
            pltpu.VMEM((2, 2, HB, 1), jnp.float32),
            pltpu.VMEM((B, H, D), jnp.float32),
            pltpu.SemaphoreType.DMA((8,)),
            pltpu.SemaphoreType.DMA(()),
            pltpu.SemaphoreType.DMA((3,)),
            pltpu.SemaphoreType.DMA((3,)),
        ],
        compiler_params=pltpu.CompilerParams(
            collective_id=0, vmem_limit_bytes=60 * 1024 * 1024),
    )(Q2, Kp, Vp, bt, lens1)

    return out[:, None, :, :]
